# baseline (speedup 1.0000x reference)
import functools

import jax
import jax.numpy as jnp
from jax import lax
from jax.experimental import pallas as pl
from jax.experimental.pallas import tpu as pltpu
from jax.experimental.pallas import tpu_sc as plsc

N = 10000
NP = 10240
E = 320000
DF = 128
DH = 16
NCLS = 40

NC = 2
NS = 16
NT = NC * NS
BW = 128
NB = 80
EP = NT * NB * BW
STRIPE = NP // NS

_MESH = plsc.VectorSubcoreMesh(core_axis_name="c", subcore_axis_name="s")


def _splat16(v, lane):
    idx = jnp.full((16, 1), lane, jnp.int32)
    dn = lax.GatherDimensionNumbers(
        offset_dims=(), collapsed_slice_dims=(0,), start_index_map=(0,))
    return lax.gather(v, idx, dn, (1,),
                      mode=lax.GatherScatterMode.PROMISE_IN_BOUNDS)


def _zero_rows(ref, nrows):
    def body(i, _):
        ref[i, :] = jnp.zeros((DH,), jnp.float32)
        return 0
    lax.fori_loop(0, nrows, body, 0)


def _agg_loop(row2d, col2d, norm2d, tbl_h, acc_sh, gbufs, sbufs, gsems, ssems):
    for j in range(3):
        pltpu.async_copy(tbl_h.at[row2d.at[j]], gbufs[j], gsems[j])

    def body(j4, _):
        for b in range(4):
            j = j4 * 4 + b
            gb, gs = gbufs[b], gsems[b]
            sb, ss = sbufs[b], ssems[b]
            pltpu.make_async_copy(tbl_h.at[row2d.at[j]], gb, gs).wait()
            nb = (b + 3) % 4
            if b == 0:
                pltpu.async_copy(tbl_h.at[row2d.at[j + 3]], gbufs[nb], gsems[nb])
            else:
                @pl.when(j4 < NB // 4 - 1)
                def _():
                    pltpu.async_copy(tbl_h.at[row2d.at[j + 3]],
                                     gbufs[nb], gsems[nb])
            @pl.when(j4 >= 1)
            def _():
                pltpu.make_async_copy(sb, acc_sh.at[col2d.at[j]], ss).wait()

            def scale(k, _2):
                nv = norm2d[j, pl.ds(k * 16, 16)]
                for l in range(16):
                    m = k * 16 + l
                    sb[m, :] = gb[m, :] * nv[l]
                return 0
            lax.fori_loop(0, BW // 16, scale, 0)

            pltpu.async_copy(sb, acc_sh.at[col2d.at[j]], ss, add=True)
        return 0

    lax.fori_loop(0, NB // 4, body, 0)
    for j in range(NB - 4, NB):
        b = j % 4
        pltpu.make_async_copy(sbufs[b], acc_sh.at[col2d.at[j]], ssems[b]).wait()


@functools.partial(
    pl.kernel,
    out_type=(
        jax.ShapeDtypeStruct((NC, NP, DH), jnp.float32),
        jax.ShapeDtypeStruct((NP // 16, 16), jnp.float32),
        jax.ShapeDtypeStruct((NT, NB, BW), jnp.float32),
    ),
    mesh=_MESH,
    compiler_params=pltpu.CompilerParams(needs_layout_passes=False, use_tc_tiling_on_sc=False),
    scratch_types=[
        pltpu.VMEM_SHARED((NP, DH), jnp.float32),
        pltpu.VMEM_SHARED((NP, DH), jnp.float32),
        pltpu.VMEM_SHARED((NP // 16, 16), jnp.float32),
        pltpu.VMEM_SHARED((NP // 16, 16), jnp.float32),
        pltpu.VMEM((NB, BW), jnp.int32),
        pltpu.VMEM((NB, BW), jnp.int32),
        pltpu.VMEM((NB, BW), jnp.float32),
        pltpu.VMEM((8, BW), jnp.int32),
        pltpu.VMEM((8, BW), jnp.float32),
        pltpu.VMEM((STRIPE // 16, 16), jnp.float32),
        pltpu.VMEM((NP // 16, 16), jnp.float32),
        pltpu.VMEM((5, BW), jnp.int32),
        pltpu.VMEM((BW, DH), jnp.float32),
        pltpu.VMEM((BW, DH), jnp.float32),
        pltpu.VMEM((BW, DH), jnp.float32),
        pltpu.VMEM((BW, DH), jnp.float32),
        pltpu.VMEM((BW, DH), jnp.float32),
        pltpu.VMEM((BW, DH), jnp.float32),
        pltpu.VMEM((BW, DH), jnp.float32),
        pltpu.VMEM((BW, DH), jnp.float32),
        pltpu.SemaphoreType.DMA,
        pltpu.SemaphoreType.DMA,
        pltpu.SemaphoreType.DMA,
        pltpu.SemaphoreType.DMA,
        pltpu.SemaphoreType.DMA,
        pltpu.SemaphoreType.DMA,
        pltpu.SemaphoreType.DMA,
        pltpu.SemaphoreType.DMA,
        pltpu.SemaphoreType.DMA,
    ],
)
def _sc_layer1(row_h, col_h, ew_h, h0_h, p_h, dis_h, norm_h,
               acc_sh, h_sh, deg_sh, dis_sh,
               row2d, col2d, ew2d, col_w, ew_w,
               degb, disf, idb, gb0, gb1, gb2, gb3, sb0, sb1, sb2, sb3,
               gsem0, gsem1, gsem2, gsem3, ssem0, ssem1, ssem2, ssem3,
               hsem):
    c = lax.axis_index("c")
    s = lax.axis_index("s")
    wid = s * 2 + c
    owid = s * 2 + (1 - c)
    st = s * STRIPE

    pltpu.sync_copy(row_h.at[wid], row2d)
    pltpu.sync_copy(col_h.at[wid], col2d)
    pltpu.sync_copy(ew_h.at[wid], ew2d)
    pltpu.sync_copy(h0_h.at[pl.ds(st, STRIPE)], h_sh.at[pl.ds(st, STRIPE)])
    _zero_rows(gb0, BW)
    for k in range(STRIPE // BW):
        pltpu.sync_copy(gb0, acc_sh.at[pl.ds(st + k * BW, BW)])
    def ones(i, _):
        degb[i, :] = jnp.full((16,), 1.0, jnp.float32)
        return 0
    lax.fori_loop(0, STRIPE // 16, ones, 0)
    pltpu.sync_copy(degb, deg_sh.at[pl.ds(s * (STRIPE // 16), STRIPE // 16)])
    def idz(g, _):
        def inner(k, _2):
            idb[g, pl.ds(k * 16, 16)] = (
                lax.iota(jnp.int32, 16) + (g * BW + k * 16))
            return 0
        lax.fori_loop(0, BW // 16, inner, 0)
        return 0
    lax.fori_loop(0, 5, idz, 0)
    def hz(i, _):
        disf[i, :] = jnp.zeros((16,), jnp.float32)
        return 0
    lax.fori_loop(0, NP // 16, hz, 0)
    plsc.subcore_barrier()

    def hist_mine(j, _):
        def inner(k, _2):
            sl = pl.ds(k * 16, 16)
            idx = col2d[j, sl]
            plsc.addupdate_scatter(disf, [idx >> 4, idx & 15], ew2d[j, sl])
            return 0
        lax.fori_loop(0, BW // 16, inner, 0)
        return 0
    lax.fori_loop(0, NB, hist_mine, 0)

    def hist_part(o, _):
        pltpu.sync_copy(col_h.at[owid].at[pl.ds(o * 8, 8)], col_w)
        pltpu.sync_copy(ew_h.at[owid].at[pl.ds(o * 8, 8)], ew_w)
        def win(i, _2):
            def inner(k, _3):
                sl = pl.ds(k * 16, 16)
                idx = col_w[i, sl]
                plsc.addupdate_scatter(disf, [idx >> 4, idx & 15],
                                       ew_w[i, sl])
                return 0
            lax.fori_loop(0, BW // 16, inner, 0)
            return 0
        lax.fori_loop(0, 8, win, 0)
        return 0
    lax.fori_loop(0, NB // 8, hist_part, 0)
    for g in range(5):
        pltpu.async_copy(disf.at[pl.ds(g * BW, BW)], deg_sh.at[idb.at[g]],
                         hsem, add=True)
    for g in range(5):
        pltpu.make_async_copy(disf.at[pl.ds(g * BW, BW)],
                              deg_sh.at[idb.at[g]], hsem).wait()
    plsc.subcore_barrier()

    nst = s * (STRIPE // 16)
    pltpu.sync_copy(deg_sh.at[pl.ds(nst, STRIPE // 16)], degb)

    def newton(v, _):
        dv = degb[v, :]
        bits = lax.bitcast_convert_type(dv, jnp.int32)
        y = lax.bitcast_convert_type(
            jnp.full((16,), 0x5F3759DF, jnp.int32) - (bits >> 1), jnp.float32)
        half = dv * 0.5
        for _i in range(4):
            y = y * (1.5 - half * y * y)
        degb[v, :] = y
        return 0
    lax.fori_loop(0, STRIPE // 16, newton, 0)
    pltpu.sync_copy(degb, dis_sh.at[pl.ds(nst, STRIPE // 16)])

    @pl.when(c == 0)
    def _():
        pltpu.sync_copy(degb, dis_h.at[pl.ds(nst, STRIPE // 16)])
    plsc.subcore_barrier()

    pltpu.sync_copy(dis_sh, disf)

    def nrm(j, _):
        def inner(k, _2):
            sl = pl.ds(k * 16, 16)
            r = row2d[j, sl]
            cc = col2d[j, sl]
            dr = plsc.load_gather(disf, [r >> 4, r & 15])
            dc = plsc.load_gather(disf, [cc >> 4, cc & 15])
            ew2d[j, sl] = dr * ew2d[j, sl] * dc
            return 0
        lax.fori_loop(0, BW // 16, inner, 0)
        return 0
    lax.fori_loop(0, NB, nrm, 0)
    pltpu.sync_copy(ew2d, norm_h.at[wid])

    _agg_loop(row2d, col2d, ew2d, h_sh, acc_sh,
              (gb0, gb1, gb2, gb3), (sb0, sb1, sb2, sb3),
              (gsem0, gsem1, gsem2, gsem3), (ssem0, ssem1, ssem2, ssem3))
    plsc.subcore_barrier()
    pltpu.sync_copy(acc_sh.at[pl.ds(st, STRIPE)],
                    p_h.at[c].at[pl.ds(st, STRIPE)])


@functools.partial(
    pl.kernel,
    out_type=jax.ShapeDtypeStruct((NC, NP, DH), jnp.float32),
    mesh=_MESH,
    compiler_params=pltpu.CompilerParams(needs_layout_passes=False, use_tc_tiling_on_sc=False),
    scratch_types=[
        pltpu.VMEM_SHARED((NP, DH), jnp.float32),
        pltpu.VMEM_SHARED((NP, DH), jnp.float32),
        pltpu.VMEM((NB, BW), jnp.int32),
        pltpu.VMEM((NB, BW), jnp.int32),
        pltpu.VMEM((NB, BW), jnp.float32),
        pltpu.VMEM((BW, DH), jnp.float32),
        pltpu.VMEM((BW, DH), jnp.float32),
        pltpu.VMEM((BW, DH), jnp.float32),
        pltpu.VMEM((BW, DH), jnp.float32),
        pltpu.VMEM((BW, DH), jnp.float32),
        pltpu.VMEM((BW, DH), jnp.float32),
        pltpu.VMEM((BW, DH), jnp.float32),
        pltpu.VMEM((BW, DH), jnp.float32),
        pltpu.SemaphoreType.DMA,
        pltpu.SemaphoreType.DMA,
        pltpu.SemaphoreType.DMA,
        pltpu.SemaphoreType.DMA,
        pltpu.SemaphoreType.DMA,
        pltpu.SemaphoreType.DMA,
        pltpu.SemaphoreType.DMA,
        pltpu.SemaphoreType.DMA,
        pltpu.SemaphoreType.DMA,
    ],
)
def _sc_layer2(row_h, col_h, norm_h, h1_h, q_h,
               acc_sh, h_sh, row2d, col2d, norm2d,
               gb0, gb1, gb2, gb3, sb0, sb1, sb2, sb3,
               gsem0, gsem1, gsem2, gsem3, ssem0, ssem1, ssem2, ssem3,
               hsem):
    c = lax.axis_index("c")
    s = lax.axis_index("s")
    wid = s * 2 + c
    st = s * STRIPE

    pltpu.sync_copy(row_h.at[wid], row2d)
    pltpu.sync_copy(col_h.at[wid], col2d)
    pltpu.sync_copy(norm_h.at[wid], norm2d)
    pltpu.sync_copy(h1_h.at[pl.ds(st, STRIPE)], h_sh.at[pl.ds(st, STRIPE)])
    _zero_rows(gb0, BW)
    for k in range(STRIPE // BW):
        pltpu.sync_copy(gb0, acc_sh.at[pl.ds(st + k * BW, BW)])
    plsc.subcore_barrier()

    _agg_loop(row2d, col2d, norm2d, h_sh, acc_sh,
              (gb0, gb1, gb2, gb3), (sb0, sb1, sb2, sb3),
              (gsem0, gsem1, gsem2, gsem3), (ssem0, ssem1, ssem2, ssem3))
    plsc.subcore_barrier()
    pltpu.sync_copy(acc_sh.at[pl.ds(st, STRIPE)],
                    q_h.at[c].at[pl.ds(st, STRIPE)])


def _mm_body(x_ref, w_ref, o_ref):
    o_ref[...] = jnp.dot(x_ref[...], w_ref[...],
                         preferred_element_type=jnp.float32)


_tc_matmul = pl.pallas_call(
    _mm_body,
    out_shape=jax.ShapeDtypeStruct((NP, DH), jnp.float32),
)


def _comb_body(p_ref, dis_ref, h0_ref, o_ref):
    d2 = dis_ref[...] * dis_ref[...]
    o_ref[...] = p_ref[0] + p_ref[1] + d2 * h0_ref[...]


_tc_combine = pl.pallas_call(
    _comb_body,
    out_shape=jax.ShapeDtypeStruct((NP, DH), jnp.float32),
)


def _final_body(q_ref, h1_ref, dis_ref, w2_ref, o_ref):
    d2 = dis_ref[...] * dis_ref[...]
    h2 = q_ref[0] + q_ref[1] + d2 * h1_ref[...]
    logits = jnp.dot(h2, w2_ref[...], preferred_element_type=jnp.float32)
    m = jnp.max(logits, axis=-1, keepdims=True)
    sh = logits - m
    lse = jnp.log(jnp.sum(jnp.exp(sh), axis=-1, keepdims=True))
    o_ref[...] = sh - lse


_tc_final = pl.pallas_call(
    _final_body,
    out_shape=jax.ShapeDtypeStruct((NP, NCLS), jnp.float32),
)


def kernel(x, edge_index, edge_weight, W1, W2):
    row = edge_index[0].astype(jnp.int32)
    col = edge_index[1].astype(jnp.int32)
    pad = EP - E
    rowp = jnp.concatenate([row, jnp.zeros((pad,), jnp.int32)]).reshape(NT, NB, BW)
    colp = jnp.concatenate([col, jnp.zeros((pad,), jnp.int32)]).reshape(NT, NB, BW)
    ewp = jnp.concatenate(
        [edge_weight.astype(jnp.float32), jnp.zeros((pad,), jnp.float32)]
    ).reshape(NT, NB, BW)
    xp = jnp.pad(x.astype(jnp.float32), ((0, NP - N), (0, 0)))

    h0 = _tc_matmul(xp, W1)
    p, dis, normv = _sc_layer1(rowp, colp, ewp, h0)
    dis2d = dis.reshape(NP, 1)
    h1 = _tc_combine(p, dis2d, h0)
    q = _sc_layer2(rowp, colp, normv, h1)
    out = _tc_final(q, h1, dis2d, W2)
    return out[:N]
